# baseline (speedup 1.0000x reference)
import functools

import jax
import jax.numpy as jnp
from jax import lax
from jax.experimental import pallas as pl
from jax.experimental.pallas import tpu as pltpu
from jax.experimental.pallas import tpu_sc as plsc

N = 10000
E = 320000
D = 128

NC = 2
NS = 16
NW = NC * NS
EPW = E // NW
K = 200
NCHUNK = EPW // K
NPAIR = NCHUNK // 2


def _mm_body(x_ref, w0_ref, w1_ref, b_ref, xr_ref, xc_ref):
    xv = x_ref[...]
    xr_ref[...] = jnp.dot(xv, w0_ref[...], preferred_element_type=jnp.float32)
    xc_ref[...] = (jnp.dot(xv, w1_ref[...], preferred_element_type=jnp.float32)
                   + b_ref[...])


def _node_tables(x, W, b):
    W0 = W[:D]
    W1 = W[D:]
    b2 = b.reshape(1, D)
    blk = 2000
    grid = N // blk
    return pl.pallas_call(
        _mm_body,
        grid=(grid,),
        in_specs=[
            pl.BlockSpec((blk, D), lambda i: (i, 0)),
            pl.BlockSpec((D, D), lambda i: (0, 0)),
            pl.BlockSpec((D, D), lambda i: (0, 0)),
            pl.BlockSpec((1, D), lambda i: (0, 0)),
        ],
        out_specs=[
            pl.BlockSpec((blk, D), lambda i: (i, 0)),
            pl.BlockSpec((blk, D), lambda i: (i, 0)),
        ],
        out_shape=[
            jax.ShapeDtypeStruct((N, D), jnp.float32),
            jax.ShapeDtypeStruct((N, D), jnp.float32),
        ],
    )(x, W0, W1, b2)


def _edge_body(xr_hbm, xc_hbm, row_hbm, col_hbm, out_hbm,
               idx_r0, idx_c0, gr0, gc0, idx_r1, idx_c1, gr1, gc1,
               sa0, sb0, sc0, sd0, sa1, sb1, sc1, sd1):
    wid = lax.axis_index("s") * NC + lax.axis_index("c")
    bufs = ((idx_r0, idx_c0, gr0, gc0, (sa0, sb0, sc0, sd0)),
            (idx_r1, idx_c1, gr1, gc1, (sa1, sb1, sc1, sd1)))

    def issue(i, b):
        idx_r, idx_c, gr, gc, sems = bufs[b]
        base = wid * EPW + i * K
        pltpu.async_copy(row_hbm.at[pl.ds(base, K)], idx_r, sems[0]).wait()
        pltpu.async_copy(col_hbm.at[pl.ds(base, K)], idx_c, sems[1]).wait()
        pltpu.async_copy(xr_hbm.at[idx_r], gr, sems[2])
        pltpu.async_copy(xc_hbm.at[idx_c], gc, sems[3])

    def finish(i, b):
        idx_r, idx_c, gr, gc, sems = bufs[b]
        base = wid * EPW + i * K
        pltpu.make_async_copy(xr_hbm.at[idx_r], gr, sems[2]).wait()
        pltpu.make_async_copy(xc_hbm.at[idx_c], gc, sems[3]).wait()

        def add_row(j):
            for cc in range(D // 16):
                sl = pl.ds(cc * 16, 16)
                gr[j, sl] = gr[j, sl] + gc[j, sl]

        plsc.parallel_loop(0, K, unroll=2)(add_row)
        pltpu.sync_copy(gr, out_hbm.at[pl.ds(base, K)])

    issue(0, 0)

    def pair(t, carry):
        i0 = t * 2
        issue(i0 + 1, 1)
        finish(i0, 0)
        issue(i0 + 2, 0)
        finish(i0 + 1, 1)
        return carry

    lax.fori_loop(0, NPAIR - 1, pair, 0)
    i0 = (NPAIR - 1) * 2
    issue(i0 + 1, 1)
    finish(i0, 0)
    finish(i0 + 1, 1)


def _edge_out(xr, xc, row, col):
    mesh = plsc.VectorSubcoreMesh(core_axis_name="c", subcore_axis_name="s")
    f = functools.partial(
        pl.kernel,
        out_type=jax.ShapeDtypeStruct((E, D), jnp.float32),
        mesh=mesh,
        scratch_types=[
            pltpu.VMEM((K,), jnp.int32),
            pltpu.VMEM((K,), jnp.int32),
            pltpu.VMEM((K, D), jnp.float32),
            pltpu.VMEM((K, D), jnp.float32),
            pltpu.VMEM((K,), jnp.int32),
            pltpu.VMEM((K,), jnp.int32),
            pltpu.VMEM((K, D), jnp.float32),
            pltpu.VMEM((K, D), jnp.float32),
        ] + [pltpu.SemaphoreType.DMA] * 8,
    )(_edge_body)
    return f(xr, xc, row, col)


NPT = 320
NPAD = NW * NPT
S = 3200
NSCAN = E // S
G = 128
CAP = 4096
NEG = -3.0e38


def _agg_body(row2_hbm, out_hbm, agg_hbm,
              rbuf, mbuf, vbuf, gbuf, aggl, sem):
    wid = lax.axis_index("s") * NC + lax.axis_index("c")
    lo = wid * NPT
    hi = lo + NPT

    neg_v = jnp.full((16,), NEG, jnp.float32)

    def init_a(i, c):
        for cc in range(D // 16):
            aggl[i, pl.ds(cc * 16, 16)] = neg_v
        return c

    lax.fori_loop(0, NPT, init_a, 0)
    zero16 = jnp.zeros((16,), jnp.int32)

    m1_16 = jnp.full((16,), -1, jnp.int32)

    def init_r(i, c):
        mbuf[pl.ds(i * 16, 16)] = zero16
        vbuf[i, :] = m1_16
        return c

    lax.fori_loop(0, CAP // 16, init_r, 0)

    iota = lax.iota(jnp.int32, 16)
    lo_v = jnp.full((16,), lo, jnp.int32)
    hi_v = jnp.full((16,), hi, jnp.int32)
    z16 = jnp.zeros((16,), jnp.int32)

    def drain_fast(rp):
        w = pl.multiple_of(rp % CAP, G)
        cp = pltpu.async_copy(out_hbm.at[mbuf.at[pl.ds(w, G)]], gbuf, sem)
        cp.wait()

        def apply16(q, c):
            vv = vbuf[(w // 16) + q, :]
            rls = [vv.at[jnp.full((16,), j2, jnp.int32)].get(
                       mode="promise_in_bounds")[0] - lo
                   for j2 in range(16)]
            for j2 in range(16):
                rl = rls[j2]
                for cc in range(D // 16):
                    sl = pl.ds(cc * 16, 16)
                    aggl[rl, sl] = jnp.maximum(aggl[rl, sl],
                                               gbuf[q * 16 + j2, sl])
            return c

        lax.fori_loop(0, G // 16, apply16, 0)
        return rp + G

    def drain(rp):
        w = pl.multiple_of(rp % CAP, G)
        cp = pltpu.async_copy(out_hbm.at[mbuf.at[pl.ds(w, G)]], gbuf, sem)
        cp.wait()

        def apply(j, c):
            vv = vbuf[(w // 16) + (j >> 4), :]
            rv = vv.at[(j & 15) + z16].get(mode="promise_in_bounds")[0]
            inr = (rv >= lo) & (rv < hi)
            rl = jnp.where(inr, rv - lo, 0)
            for cc in range(D // 16):
                sl = pl.ds(cc * 16, 16)
                val = jnp.where(inr, gbuf[j, sl], neg_v)
                aggl[rl, sl] = jnp.maximum(aggl[rl, sl], val)
            return c

        lax.fori_loop(0, G, apply, 0)
        return rp + G

    def chunk(i, state):
        ofs_v, rp = state
        cbase = i * S
        pltpu.sync_copy(row2_hbm.at[pl.ds(i * (S // 16), S // 16)], rbuf)

        def group(g, ofs_v):
            v = rbuf[g, :]
            msk = (v >= lo_v) & (v < hi_v)
            mi = msk.astype(jnp.int32)
            ids = jnp.full((16,), cbase, jnp.int32) + g * 16 + iota
            pos = (ofs_v + plsc.cumsum(mi) - mi) % CAP
            plsc.store_scatter(mbuf, [pos], ids, mask=msk)
            plsc.store_scatter(vbuf, [pos >> 4, pos & 15], v, mask=msk)
            return ofs_v + plsc.all_reduce_population_count(msk)

        ofs_v = plsc.parallel_loop(0, S // 16, unroll=16, carry=ofs_v)(group)
        o = ofs_v[0]
        nd = (o - rp) // G
        rp = lax.fori_loop(0, nd, lambda k, r: drain_fast(r), rp)
        return ofs_v, rp

    ofs_v, rp = lax.fori_loop(
        0, NSCAN, chunk, (jnp.zeros((16,), jnp.int32), jnp.int32(0)))
    rp = drain(rp)

    zf = jnp.zeros((16,), jnp.float32)

    def fix(i, c):
        for cc in range(D // 16):
            sl = pl.ds(cc * 16, 16)
            a = aggl[i, sl]
            aggl[i, sl] = jnp.where(a <= neg_v, zf, a)
        return c

    lax.fori_loop(0, NPT, fix, 0)
    pltpu.sync_copy(aggl, agg_hbm.at[pl.ds(lo, NPT)])


def _segment_max(row, out):
    mesh = plsc.VectorSubcoreMesh(core_axis_name="c", subcore_axis_name="s")
    f = functools.partial(
        pl.kernel,
        out_type=jax.ShapeDtypeStruct((NPAD, D), jnp.float32),
        mesh=mesh,
        scratch_types=[
            pltpu.VMEM((S // 16, 16), jnp.int32),
            pltpu.VMEM((CAP,), jnp.int32),
            pltpu.VMEM((CAP // 16, 16), jnp.int32),
            pltpu.VMEM((G, D), jnp.float32),
            pltpu.VMEM((NPT, D), jnp.float32),
            pltpu.SemaphoreType.DMA,
        ],
        compiler_params=pltpu.CompilerParams(needs_layout_passes=False),
    )(_agg_body)
    return f(row.reshape(E // 16, 16), out)[:N]


def kernel(x, edge_index, W, b):
    row = edge_index[0]
    col = edge_index[1]
    xr, xc = _node_tables(x, W, b)
    out = _edge_out(xr, xc, row, col)
    agg = _segment_max(row, out)
    return (agg, out)


# speedup vs baseline: 1.0190x; 1.0190x over previous
import functools

import jax
import jax.numpy as jnp
from jax import lax
from jax.experimental import pallas as pl
from jax.experimental.pallas import tpu as pltpu
from jax.experimental.pallas import tpu_sc as plsc

N = 10000
E = 320000
D = 128

NC = 2
NS = 16
NW = NC * NS
EPW = E // NW
K = 200
NCHUNK = EPW // K
NPAIR = NCHUNK // 2


def _mm_body(x_ref, w0_ref, w1_ref, b_ref, xr_ref, xc_ref):
    xv = x_ref[...]
    xr_ref[...] = jnp.dot(xv, w0_ref[...], preferred_element_type=jnp.float32)
    xc_ref[...] = (jnp.dot(xv, w1_ref[...], preferred_element_type=jnp.float32)
                   + b_ref[...])


def _node_tables(x, W, b):
    W0 = W[:D]
    W1 = W[D:]
    b2 = b.reshape(1, D)
    blk = 2000
    grid = N // blk
    return pl.pallas_call(
        _mm_body,
        grid=(grid,),
        in_specs=[
            pl.BlockSpec((blk, D), lambda i: (i, 0)),
            pl.BlockSpec((D, D), lambda i: (0, 0)),
            pl.BlockSpec((D, D), lambda i: (0, 0)),
            pl.BlockSpec((1, D), lambda i: (0, 0)),
        ],
        out_specs=[
            pl.BlockSpec((blk, D), lambda i: (i, 0)),
            pl.BlockSpec((blk, D), lambda i: (i, 0)),
        ],
        out_shape=[
            jax.ShapeDtypeStruct((N, D), jnp.float32),
            jax.ShapeDtypeStruct((N, D), jnp.float32),
        ],
    )(x, W0, W1, b2)


def _edge_body(xr_hbm, xc_hbm, row_hbm, col_hbm, out_hbm,
               idx_r0, idx_c0, gr0, gc0, idx_r1, idx_c1, gr1, gc1,
               sa0, sb0, sc0, sd0, sa1, sb1, sc1, sd1):
    wid = lax.axis_index("s") * NC + lax.axis_index("c")
    bufs = ((idx_r0, idx_c0, gr0, gc0, (sa0, sb0, sc0, sd0)),
            (idx_r1, idx_c1, gr1, gc1, (sa1, sb1, sc1, sd1)))

    def issue(i, b):
        idx_r, idx_c, gr, gc, sems = bufs[b]
        base = wid * EPW + i * K
        pltpu.async_copy(row_hbm.at[pl.ds(base, K)], idx_r, sems[0]).wait()
        pltpu.async_copy(col_hbm.at[pl.ds(base, K)], idx_c, sems[1]).wait()
        pltpu.async_copy(xr_hbm.at[idx_r], gr, sems[2])
        pltpu.async_copy(xc_hbm.at[idx_c], gc, sems[3])

    def finish(i, b):
        idx_r, idx_c, gr, gc, sems = bufs[b]
        base = wid * EPW + i * K
        pltpu.make_async_copy(xr_hbm.at[idx_r], gr, sems[2]).wait()
        pltpu.make_async_copy(xc_hbm.at[idx_c], gc, sems[3]).wait()

        def add_row(j):
            for cc in range(D // 16):
                sl = pl.ds(cc * 16, 16)
                gr[j, sl] = gr[j, sl] + gc[j, sl]

        plsc.parallel_loop(0, K, unroll=2)(add_row)
        pltpu.sync_copy(gr, out_hbm.at[pl.ds(base, K)])

    issue(0, 0)

    def pair(t, carry):
        i0 = t * 2
        issue(i0 + 1, 1)
        finish(i0, 0)
        issue(i0 + 2, 0)
        finish(i0 + 1, 1)
        return carry

    lax.fori_loop(0, NPAIR - 1, pair, 0)
    i0 = (NPAIR - 1) * 2
    issue(i0 + 1, 1)
    finish(i0, 0)
    finish(i0 + 1, 1)


def _edge_out(xr, xc, row, col):
    mesh = plsc.VectorSubcoreMesh(core_axis_name="c", subcore_axis_name="s")
    f = functools.partial(
        pl.kernel,
        out_type=jax.ShapeDtypeStruct((E, D), jnp.float32),
        mesh=mesh,
        scratch_types=[
            pltpu.VMEM((K,), jnp.int32),
            pltpu.VMEM((K,), jnp.int32),
            pltpu.VMEM((K, D), jnp.float32),
            pltpu.VMEM((K, D), jnp.float32),
            pltpu.VMEM((K,), jnp.int32),
            pltpu.VMEM((K,), jnp.int32),
            pltpu.VMEM((K, D), jnp.float32),
            pltpu.VMEM((K, D), jnp.float32),
        ] + [pltpu.SemaphoreType.DMA] * 8,
    )(_edge_body)
    return f(xr, xc, row, col)


NPT = 320
NPAD = NW * NPT
S = 3200
NSCAN = E // S
G = 128
CAP = 4096
NEG = -3.0e38


def _agg_body(row2_hbm, out_hbm, agg_hbm,
              rbuf, mbuf, vbuf, gbuf, aggl, sem):
    wid = lax.axis_index("s") * NC + lax.axis_index("c")
    lo = wid * NPT
    hi = lo + NPT

    neg_v = jnp.full((16,), NEG, jnp.float32)

    def init_a(i, c):
        for cc in range(D // 16):
            aggl[i, pl.ds(cc * 16, 16)] = neg_v
        return c

    lax.fori_loop(0, NPT, init_a, 0)
    zero16 = jnp.zeros((16,), jnp.int32)

    m1_16 = jnp.full((16,), -1, jnp.int32)

    def init_r(i, c):
        mbuf[pl.ds(i * 16, 16)] = zero16
        vbuf[i, :] = m1_16
        return c

    lax.fori_loop(0, CAP // 16, init_r, 0)

    iota = lax.iota(jnp.int32, 16)
    lo_v = jnp.full((16,), lo, jnp.int32)
    hi_v = jnp.full((16,), hi, jnp.int32)
    z16 = jnp.zeros((16,), jnp.int32)

    def drain_fast(rp):
        w = pl.multiple_of(rp % CAP, G)
        cp = pltpu.async_copy(out_hbm.at[mbuf.at[pl.ds(w, G)]], gbuf, sem)
        cp.wait()

        def apply16(q, c):
            vv = vbuf[(w // 16) + q, :]
            for j2 in range(16):
                rv = vv.at[jnp.full((16,), j2, jnp.int32)].get(
                    mode="promise_in_bounds")[0]
                rl = rv - lo
                for cc in range(D // 16):
                    sl = pl.ds(cc * 16, 16)
                    aggl[rl, sl] = jnp.maximum(aggl[rl, sl],
                                               gbuf[q * 16 + j2, sl])
            return c

        lax.fori_loop(0, G // 16, apply16, 0)
        return rp + G

    def drain(rp):
        w = pl.multiple_of(rp % CAP, G)
        cp = pltpu.async_copy(out_hbm.at[mbuf.at[pl.ds(w, G)]], gbuf, sem)
        cp.wait()

        def apply(j, c):
            vv = vbuf[(w // 16) + (j >> 4), :]
            rv = vv.at[(j & 15) + z16].get(mode="promise_in_bounds")[0]
            inr = (rv >= lo) & (rv < hi)
            rl = jnp.where(inr, rv - lo, 0)
            for cc in range(D // 16):
                sl = pl.ds(cc * 16, 16)
                val = jnp.where(inr, gbuf[j, sl], neg_v)
                aggl[rl, sl] = jnp.maximum(aggl[rl, sl], val)
            return c

        lax.fori_loop(0, G, apply, 0)
        return rp + G

    def chunk(i, state):
        ofs_v, rp = state
        cbase = i * S
        pltpu.sync_copy(row2_hbm.at[pl.ds(i * (S // 16), S // 16)], rbuf)

        def group(g, ofs_v):
            v = rbuf[g, :]
            msk = (v >= lo_v) & (v < hi_v)
            mi = msk.astype(jnp.int32)
            ids = jnp.full((16,), cbase, jnp.int32) + g * 16 + iota
            pos = (ofs_v + plsc.cumsum(mi) - mi) % CAP
            plsc.store_scatter(mbuf, [pos], ids, mask=msk)
            plsc.store_scatter(vbuf, [pos >> 4, pos & 15], v, mask=msk)
            return ofs_v + plsc.all_reduce_population_count(msk)

        ofs_v = plsc.parallel_loop(0, S // 16, unroll=8, carry=ofs_v)(group)
        o = ofs_v[0]
        nd = (o - rp) // G
        rp = lax.fori_loop(0, nd, lambda k, r: drain_fast(r), rp)
        return ofs_v, rp

    ofs_v, rp = lax.fori_loop(
        0, NSCAN, chunk, (jnp.zeros((16,), jnp.int32), jnp.int32(0)))
    rp = drain(rp)

    zf = jnp.zeros((16,), jnp.float32)

    def fix(i, c):
        for cc in range(D // 16):
            sl = pl.ds(cc * 16, 16)
            a = aggl[i, sl]
            aggl[i, sl] = jnp.where(a <= neg_v, zf, a)
        return c

    lax.fori_loop(0, NPT, fix, 0)
    pltpu.sync_copy(aggl, agg_hbm.at[pl.ds(lo, NPT)])


def _segment_max(row, out):
    mesh = plsc.VectorSubcoreMesh(core_axis_name="c", subcore_axis_name="s")
    f = functools.partial(
        pl.kernel,
        out_type=jax.ShapeDtypeStruct((NPAD, D), jnp.float32),
        mesh=mesh,
        scratch_types=[
            pltpu.VMEM((S // 16, 16), jnp.int32),
            pltpu.VMEM((CAP,), jnp.int32),
            pltpu.VMEM((CAP // 16, 16), jnp.int32),
            pltpu.VMEM((G, D), jnp.float32),
            pltpu.VMEM((NPT, D), jnp.float32),
            pltpu.SemaphoreType.DMA,
        ],
        compiler_params=pltpu.CompilerParams(needs_layout_passes=False),
    )(_agg_body)
    return f(row.reshape(E // 16, 16), out)[:N]


def kernel(x, edge_index, W, b):
    row = edge_index[0]
    col = edge_index[1]
    xr, xc = _node_tables(x, W, b)
    out = _edge_out(xr, xc, row, col)
    agg = _segment_max(row, out)
    return (agg, out)
